# baseline (speedup 1.0000x reference)
import functools

import jax
import jax.numpy as jnp
from jax import lax
from jax.experimental import pallas as pl
from jax.experimental.pallas import tpu as pltpu
from jax.experimental.pallas import tpu_sc as plsc

N = 50000
E = 800000
D_IN = 128
H = 64
HQ = 16
C = 8
P_PER = 64
P = C * P_PER
E_INTER = 256

EPAD = 819200
IDXROWS = EPAD // 128
NPAD = 50176
DUMMY = NPAD
ACC_ROWS = 51200
NTILES = 16
ROWS_PER_TILE = IDXROWS // NTILES
NBUF = 2
GRP = 8
NSG = ROWS_PER_TILE // GRP
INIT_PER_TILE = NPAD // NTILES
INIT_CHUNK = 784

NB = 25
BLK = N // NB


@functools.lru_cache(maxsize=None)
def _sc_mesh():
    return plsc.VectorSubcoreMesh(core_axis_name="c", subcore_axis_name="s",
                                  num_cores=2, num_subcores=16)


def _sc_scatter(u0_hbm, u1_hbm, u2_hbm, u3_hbm, src_hbm, dst_hbm,
                s0_hbm, s1_hbm, s2_hbm, s3_hbm,
                si0, si1, si2, si3, di0, di1, di2, di3,
                rv0, rv1, rv2, rv3, ibuf, acc, gsem, ssem):
    src_i = [si0, si1, si2, si3]
    dst_i = [di0, di1, di2, di3]
    rows_v = [rv0, rv1, rv2, rv3]
    c = lax.axis_index("c")
    sid = lax.axis_index("s")
    r0 = sid * INIT_PER_TILE
    row0 = sid * ROWS_PER_TILE

    def chunk_pass(u_hbm, out_hbm):
        for k in range(INIT_PER_TILE // INIT_CHUNK):
            sl = pl.ds(r0 + k * INIT_CHUNK, INIT_CHUNK)
            pltpu.sync_copy(u_hbm.at[sl], ibuf)
            pltpu.sync_copy(ibuf, acc.at[sl])
        plsc.subcore_barrier()

        def fire_g(b, sg):
            base = row0 + sg * GRP
            pltpu.sync_copy(src_hbm.at[pl.ds(base, GRP)], src_i[b])
            pltpu.sync_copy(dst_hbm.at[pl.ds(base, GRP)], dst_i[b])
            for j in range(GRP):
                pltpu.async_copy(u_hbm.at[src_i[b].at[j]], rows_v[b].at[j],
                                 gsem.at[b])

        def wait_g(b):
            for j in range(GRP):
                pltpu.make_async_copy(u_hbm.at[src_i[b].at[j]],
                                      rows_v[b].at[j], gsem.at[b]).wait()

        def fire_s(b):
            for j in range(GRP):
                pltpu.async_copy(rows_v[b].at[j], acc.at[dst_i[b].at[j]],
                                 ssem.at[b], add=True)

        def drain_s(b):
            for j in range(GRP):
                pltpu.make_async_copy(rows_v[b].at[j],
                                      acc.at[dst_i[b].at[j]],
                                      ssem.at[b]).wait()

        fire_g(0, 0)

        def pair(p, carry):
            wait_g(0)
            fire_s(0)
            fire_g(1, 2 * p + 1)
            drain_s(0)
            wait_g(1)
            fire_s(1)
            fire_g(0, jnp.minimum(2 * p + 2, NSG - 1))
            drain_s(1)
            return carry

        lax.fori_loop(0, NSG // 2, pair, 0)
        wait_g(0)
        plsc.subcore_barrier()
        for k in range(INIT_PER_TILE // INIT_CHUNK):
            sl = pl.ds(r0 + k * INIT_CHUNK, INIT_CHUNK)
            pltpu.sync_copy(acc.at[sl], ibuf)
            pltpu.sync_copy(ibuf, out_hbm.at[sl])
        plsc.subcore_barrier()

    @pl.when(c == 0)
    def _():
        chunk_pass(u0_hbm, s0_hbm)
        chunk_pass(u1_hbm, s1_hbm)

    @pl.when(c == 1)
    def _():
        chunk_pass(u2_hbm, s2_hbm)
        chunk_pass(u3_hbm, s3_hbm)


DEG_ROWS_PER_TILE = IDXROWS // 32


def _sc_degree(ones_hbm, dst_hbm, dga_hbm, dgb_hbm, di0, di1, ibuf, acc, ssem):
    c = lax.axis_index("c")
    sid = lax.axis_index("s")
    r0 = sid * INIT_PER_TILE
    pltpu.sync_copy(ones_hbm, ibuf)
    for k in range(INIT_PER_TILE // INIT_CHUNK):
        pltpu.sync_copy(ibuf, acc.at[pl.ds(r0 + k * INIT_CHUNK, INIT_CHUNK)])
    plsc.subcore_barrier()

    dst_i = [di0, di1]
    row0 = (c * NTILES + sid) * DEG_ROWS_PER_TILE

    def fire(b, sg):
        base = row0 + sg * GRP
        pltpu.sync_copy(dst_hbm.at[pl.ds(base, GRP)], dst_i[b])
        for j in range(GRP):
            pltpu.async_copy(ibuf.at[pl.ds(0, 128)], acc.at[dst_i[b].at[j]],
                             ssem.at[b], add=True)

    def drain(b):
        for j in range(GRP):
            pltpu.make_async_copy(ibuf.at[pl.ds(0, 128)],
                                  acc.at[dst_i[b].at[j]], ssem.at[b]).wait()

    fire(0, 0)

    def pair(p, carry):
        fire(1, 2 * p + 1)
        drain(0)
        fire(0, 2 * p + 2)
        drain(1)
        return carry

    lax.fori_loop(0, (DEG_ROWS_PER_TILE // GRP - 1) // 2, pair, 0)
    drain(0)
    plsc.subcore_barrier()

    def copy_out(out_ref):
        for k in range(INIT_PER_TILE // INIT_CHUNK):
            sl = pl.ds(r0 + k * INIT_CHUNK, INIT_CHUNK)
            pltpu.sync_copy(acc.at[sl], ibuf)
            pltpu.sync_copy(ibuf, out_ref.at[sl])

    @pl.when(c == 0)
    def _():
        copy_out(dga_hbm)

    @pl.when(c == 1)
    def _():
        copy_out(dgb_hbm)


@functools.lru_cache(maxsize=None)
def _deg_call():
    qs = jax.ShapeDtypeStruct((NPAD, HQ), jnp.float32)
    return pl.kernel(
        _sc_degree,
        out_type=[qs, qs],
        mesh=_sc_mesh(),
        compiler_params=pltpu.CompilerParams(use_tc_tiling_on_sc=False),
        scratch_types=[
            pltpu.VMEM((GRP, 128), jnp.int32),
            pltpu.VMEM((GRP, 128), jnp.int32),
            pltpu.VMEM((INIT_CHUNK, HQ), jnp.float32),
            pltpu.VMEM_SHARED((ACC_ROWS, HQ), jnp.float32),
            pltpu.SemaphoreType.DMA((2,)),
        ],
    )


@functools.lru_cache(maxsize=None)
def _scatter_call():
    qs = jax.ShapeDtypeStruct((NPAD, HQ), jnp.float32)
    return pl.kernel(
        _sc_scatter,
        out_type=[qs, qs, qs, qs],
        mesh=_sc_mesh(),
        compiler_params=pltpu.CompilerParams(use_tc_tiling_on_sc=False),
        scratch_types=(
            [pltpu.VMEM((GRP, 128), jnp.int32)] * 8
            + [pltpu.VMEM((GRP, 128, HQ), jnp.float32)] * 4
            + [
                pltpu.VMEM((INIT_CHUNK, HQ), jnp.float32),
                pltpu.VMEM_SHARED((ACC_ROWS, HQ), jnp.float32),
                pltpu.SemaphoreType.DMA((NBUF,)),
                pltpu.SemaphoreType.DMA((NBUF,)),
            ]
        ),
    )


def _tc_first(dga_ref, dgb_ref, x_ref, w_ref, u0_ref, u1_ref, u2_ref, u3_ref,
              dinv_ref):
    deg = dga_ref[:, 0] + dgb_ref[:, 0] - 1.0
    dinv = (1.0 / jnp.sqrt(deg))[:, None]
    t = jnp.dot(x_ref[...], w_ref[...], preferred_element_type=jnp.float32)
    u = t * dinv
    u0_ref[...] = u[:, 0 * HQ:1 * HQ]
    u1_ref[...] = u[:, 1 * HQ:2 * HQ]
    u2_ref[...] = u[:, 2 * HQ:3 * HQ]
    u3_ref[...] = u[:, 3 * HQ:4 * HQ]
    dinv_ref[...] = dinv


def _tc_layer(s0_ref, s1_ref, s2_ref, s3_ref, dinv_ref, b_ref, w_ref,
              u0_ref, u1_ref, u2_ref, u3_ref):
    dinv = dinv_ref[...]
    spu = jnp.concatenate(
        [s0_ref[...], s1_ref[...], s2_ref[...], s3_ref[...]], axis=1)
    h = jnp.maximum(spu * dinv + b_ref[...], 0.0)
    u = jnp.dot(h, w_ref[...], preferred_element_type=jnp.float32) * dinv
    u0_ref[...] = u[:, 0 * HQ:1 * HQ]
    u1_ref[...] = u[:, 1 * HQ:2 * HQ]
    u2_ref[...] = u[:, 2 * HQ:3 * HQ]
    u3_ref[...] = u[:, 3 * HQ:4 * HQ]


def _tc_pool(s0_ref, s1_ref, s2_ref, s3_ref, dinv_ref, b_ref, batch_ref,
             sums_ref, cnt_ref):
    i = pl.program_id(0)

    @pl.when(i == 0)
    def _():
        sums_ref[...] = jnp.zeros_like(sums_ref)
        cnt_ref[...] = jnp.zeros_like(cnt_ref)

    dinv = dinv_ref[...]
    spu = jnp.concatenate(
        [s0_ref[...], s1_ref[...], s2_ref[...], s3_ref[...]], axis=1)
    h = jnp.maximum(spu * dinv + b_ref[...], 0.0)
    seg = lax.broadcasted_iota(jnp.int32, (P, BLK), 0)
    onehot = (seg == batch_ref[0, 0, :][None, :]).astype(jnp.float32)
    sums_ref[...] += jnp.dot(onehot, h, preferred_element_type=jnp.float32)
    cnt_ref[...] += jnp.sum(onehot, axis=1, keepdims=True) * jnp.ones(
        (1, 8), jnp.float32)


def _tc_stage2(sums_ref, cnt_ref, iei_ref,
               wi1_ref, bi1_ref, wi2_ref, bi2_ref,
               wp1_ref, bp1_ref, wp2_ref, bp2_ref, out_ref):
    crs = []
    for c in range(C):
        poly = (sums_ref[c * P_PER:(c + 1) * P_PER, :]
                / jnp.maximum(cnt_ref[c * P_PER:(c + 1) * P_PER, :1], 1.0))
        src = iei_ref[c, 0, :]
        dst = iei_ref[c, 1, :]
        io = lax.broadcasted_iota(jnp.int32, (P_PER, E_INTER), 0)
        oh_s = (io == src[None, :]).astype(jnp.float32)
        oh_d = (io == dst[None, :]).astype(jnp.float32)
        adj = lax.dot_general(oh_d, oh_s, (((1,), (1,)), ((), ())),
                              preferred_element_type=jnp.float32)
        deg = 1.0 + jnp.sum(adj, axis=1)
        dv = (1.0 / jnp.sqrt(deg))[:, None]
        nadj = dv * adj * dv[:, 0][None, :] + jnp.diag((dv * dv)[:, 0])

        def conv(px, w_ref, b_ref):
            t = jnp.dot(px, w_ref[...], preferred_element_type=jnp.float32)
            return jnp.dot(nadj, t, preferred_element_type=jnp.float32) + b_ref[...]

        h = jnp.maximum(conv(poly, wi1_ref, bi1_ref), 0.0)
        h = jnp.maximum(conv(h, wi2_ref, bi2_ref), 0.0)
        crs.append(jnp.mean(h, axis=0)[None, :])
    cr = jnp.concatenate(crs, axis=0)
    z = jnp.maximum(jnp.dot(cr, wp1_ref[...],
                            preferred_element_type=jnp.float32) + bp1_ref[...],
                    0.0)
    out_ref[...] = jnp.dot(z, wp2_ref[...],
                           preferred_element_type=jnp.float32) + bp2_ref[...]


def _row_spec(shape):
    return pl.BlockSpec(shape, lambda i: (i, 0))


def _full_spec(shape):
    return pl.BlockSpec(shape, lambda i: (0, 0))


_q_blocks = [_row_spec((BLK, HQ)) for _ in range(4)]
_q_shapes = [jax.ShapeDtypeStruct((NPAD, HQ), jnp.float32) for _ in range(4)]

_first_call = pl.pallas_call(
    _tc_first,
    grid=(NB,),
    in_specs=[_row_spec((BLK, HQ)), _row_spec((BLK, HQ)),
              _row_spec((BLK, D_IN)), _full_spec((D_IN, H))],
    out_specs=_q_blocks + [_row_spec((BLK, 1))],
    out_shape=_q_shapes + [jax.ShapeDtypeStruct((N, 1), jnp.float32)],
)

_layer_call = pl.pallas_call(
    _tc_layer,
    grid=(NB,),
    in_specs=_q_blocks + [_row_spec((BLK, 1)), _full_spec((1, H)),
                          _full_spec((H, H))],
    out_specs=_q_blocks,
    out_shape=_q_shapes,
)

_pool_call = pl.pallas_call(
    _tc_pool,
    grid=(NB,),
    in_specs=_q_blocks + [_row_spec((BLK, 1)), _full_spec((1, H)),
                          pl.BlockSpec((1, 1, BLK), lambda i: (i, 0, 0))],
    out_specs=[_full_spec((P, H)), _full_spec((P, 8))],
    out_shape=[jax.ShapeDtypeStruct((P, H), jnp.float32),
               jax.ShapeDtypeStruct((P, 8), jnp.float32)],
)

_stage2_call = pl.pallas_call(
    _tc_stage2,
    grid=(1,),
    in_specs=[_full_spec((P, H)), _full_spec((P, 8)),
              pl.BlockSpec((C, 2, E_INTER), lambda i: (0, 0, 0)),
              _full_spec((H, H)), _full_spec((1, H)),
              _full_spec((H, H)), _full_spec((1, H)),
              _full_spec((H, H)), _full_spec((1, H)),
              _full_spec((H, 1)), _full_spec((1, 1))],
    out_specs=[_full_spec((C, 1))],
    out_shape=[jax.ShapeDtypeStruct((C, 1), jnp.float32)],
)


@jax.jit
def kernel(x, edge_index, batch, inter_edge_index,
           W_intra1, b_intra1, W_intra2, b_intra2, W_intra3, b_intra3,
           W_inter1, b_inter1, W_inter2, b_inter2,
           W_p1, b_p1, W_p2, b_p2):
    pad = EPAD - E
    src = jnp.concatenate([edge_index[0],
                           jnp.zeros((pad,), jnp.int32)]).reshape(IDXROWS, 128)
    dst = jnp.concatenate([edge_index[1],
                           jnp.full((pad,), DUMMY, jnp.int32)]).reshape(
                               IDXROWS, 128)
    ones_i = jnp.ones((INIT_CHUNK, HQ), jnp.float32)

    dga, dgb = _deg_call()(ones_i, dst)
    *u, dinv = _first_call(dga, dgb, x, W_intra1)
    s = _scatter_call()(*u, src, dst)
    u = _layer_call(*s, dinv, b_intra1.reshape(1, H), W_intra2)
    s = _scatter_call()(*u, src, dst)
    u = _layer_call(*s, dinv, b_intra2.reshape(1, H), W_intra3)
    s = _scatter_call()(*u, src, dst)
    sums, cnt = _pool_call(*s, dinv, b_intra3.reshape(1, H),
                           batch.reshape(NB, 1, BLK))
    pred = _stage2_call(
        sums, cnt, inter_edge_index,
        W_inter1, b_inter1.reshape(1, H), W_inter2, b_inter2.reshape(1, H),
        W_p1, b_p1.reshape(1, H), W_p2, b_p2.reshape(1, 1))
    return pred[0]


# speedup vs baseline: 1.0310x; 1.0310x over previous
import functools

import jax
import jax.numpy as jnp
from jax import lax
from jax.experimental import pallas as pl
from jax.experimental.pallas import tpu as pltpu
from jax.experimental.pallas import tpu_sc as plsc

N = 50000
E = 800000
D_IN = 128
H = 64
HQ = 16
C = 8
P_PER = 64
P = C * P_PER
E_INTER = 256

EPAD = 819200
IDXROWS = EPAD // 128
NPAD = 50176
DUMMY = NPAD
ACC_ROWS = 51200
NTILES = 16
ROWS_PER_TILE = IDXROWS // NTILES
NBUF = 2
GRP = 8
NSG = ROWS_PER_TILE // GRP
INIT_PER_TILE = NPAD // NTILES
INIT_CHUNK = 784

NB = 25
BLK = N // NB


@functools.lru_cache(maxsize=None)
def _sc_mesh():
    return plsc.VectorSubcoreMesh(core_axis_name="c", subcore_axis_name="s",
                                  num_cores=2, num_subcores=16)


def _sc_scatter(u0_hbm, u1_hbm, u2_hbm, u3_hbm, edges_hbm,
                s0_hbm, s1_hbm, s2_hbm, s3_hbm,
                ib0, ib1, rv0, rv1, ibuf, acc, isem, gsem, ssem):
    ib = [ib0, ib1]
    rows_v = [rv0, rv1]
    c = lax.axis_index("c")
    sid = lax.axis_index("s")
    r0 = sid * INIT_PER_TILE
    row0 = sid * ROWS_PER_TILE

    def chunk_pass(u_hbm, out_hbm):
        for k in range(INIT_PER_TILE // INIT_CHUNK):
            sl = pl.ds(r0 + k * INIT_CHUNK, INIT_CHUNK)
            pltpu.sync_copy(u_hbm.at[sl], ibuf)
            pltpu.sync_copy(ibuf, acc.at[sl])
        plsc.subcore_barrier()

        def fire_i(b, sg):
            base = row0 + sg * GRP
            pltpu.async_copy(edges_hbm.at[pl.ds(base, GRP)], ib[b], isem.at[b])

        def wait_i(b, sg):
            base = row0 + sg * GRP
            pltpu.make_async_copy(edges_hbm.at[pl.ds(base, GRP)], ib[b],
                                  isem.at[b]).wait()

        def fire_g(b):
            for j in range(GRP):
                pltpu.async_copy(u_hbm.at[ib[b].at[j, 0]], rows_v[b].at[j],
                                 gsem.at[b])

        def wait_g(b):
            for j in range(GRP):
                pltpu.make_async_copy(u_hbm.at[ib[b].at[j, 0]],
                                      rows_v[b].at[j], gsem.at[b]).wait()

        def fire_s(b):
            for j in range(GRP):
                pltpu.async_copy(rows_v[b].at[j], acc.at[ib[b].at[j, 1]],
                                 ssem.at[b], add=True)

        def drain_s(b):
            for j in range(GRP):
                pltpu.make_async_copy(rows_v[b].at[j],
                                      acc.at[ib[b].at[j, 1]],
                                      ssem.at[b]).wait()

        def load_i(b, sg):
            fire_i(b, sg)
            wait_i(b, sg)

        load_i(0, 0)
        fire_g(0)

        def pair(p, carry):
            a2 = jnp.minimum(2 * p + 2, NSG - 1)
            wait_g(0)
            fire_s(0)
            load_i(1, 2 * p + 1)
            fire_g(1)
            drain_s(0)
            wait_g(1)
            fire_s(1)
            load_i(0, a2)
            fire_g(0)
            drain_s(1)
            return carry

        lax.fori_loop(0, NSG // 2, pair, 0)
        wait_g(0)
        plsc.subcore_barrier()
        for k in range(INIT_PER_TILE // INIT_CHUNK):
            sl = pl.ds(r0 + k * INIT_CHUNK, INIT_CHUNK)
            pltpu.sync_copy(acc.at[sl], ibuf)
            pltpu.sync_copy(ibuf, out_hbm.at[sl])
        plsc.subcore_barrier()

    @pl.when(c == 0)
    def _():
        chunk_pass(u0_hbm, s0_hbm)
        chunk_pass(u1_hbm, s1_hbm)

    @pl.when(c == 1)
    def _():
        chunk_pass(u2_hbm, s2_hbm)
        chunk_pass(u3_hbm, s3_hbm)


DEG_ROWS_PER_TILE = IDXROWS // 32


def _sc_degree(ones_hbm, edges_hbm, dga_hbm, dgb_hbm, di0, di1, ibuf, acc,
               ssem):
    c = lax.axis_index("c")
    sid = lax.axis_index("s")
    r0 = sid * INIT_PER_TILE
    pltpu.sync_copy(ones_hbm, ibuf)
    for k in range(INIT_PER_TILE // INIT_CHUNK):
        pltpu.sync_copy(ibuf, acc.at[pl.ds(r0 + k * INIT_CHUNK, INIT_CHUNK)])
    plsc.subcore_barrier()

    dst_i = [di0, di1]
    row0 = (c * NTILES + sid) * DEG_ROWS_PER_TILE

    def fire(b, sg):
        base = row0 + sg * GRP
        pltpu.sync_copy(edges_hbm.at[pl.ds(base, GRP)], dst_i[b])
        for j in range(GRP):
            pltpu.async_copy(ibuf.at[pl.ds(0, 128)],
                             acc.at[dst_i[b].at[j, 1]],
                             ssem.at[b], add=True)

    def drain(b):
        for j in range(GRP):
            pltpu.make_async_copy(ibuf.at[pl.ds(0, 128)],
                                  acc.at[dst_i[b].at[j, 1]],
                                  ssem.at[b]).wait()

    fire(0, 0)

    def pair(p, carry):
        fire(1, 2 * p + 1)
        drain(0)
        fire(0, 2 * p + 2)
        drain(1)
        return carry

    lax.fori_loop(0, (DEG_ROWS_PER_TILE // GRP - 1) // 2, pair, 0)
    drain(0)
    plsc.subcore_barrier()

    def copy_out(out_ref):
        for k in range(INIT_PER_TILE // INIT_CHUNK):
            sl = pl.ds(r0 + k * INIT_CHUNK, INIT_CHUNK)
            pltpu.sync_copy(acc.at[sl], ibuf)
            pltpu.sync_copy(ibuf, out_ref.at[sl])

    @pl.when(c == 0)
    def _():
        copy_out(dga_hbm)

    @pl.when(c == 1)
    def _():
        copy_out(dgb_hbm)


@functools.lru_cache(maxsize=None)
def _deg_call():
    qs = jax.ShapeDtypeStruct((NPAD, HQ), jnp.float32)
    return pl.kernel(
        _sc_degree,
        out_type=[qs, qs],
        mesh=_sc_mesh(),
        compiler_params=pltpu.CompilerParams(use_tc_tiling_on_sc=False),
        scratch_types=[
            pltpu.VMEM((GRP, 2, 128), jnp.int32),
            pltpu.VMEM((GRP, 2, 128), jnp.int32),
            pltpu.VMEM((INIT_CHUNK, HQ), jnp.float32),
            pltpu.VMEM_SHARED((ACC_ROWS, HQ), jnp.float32),
            pltpu.SemaphoreType.DMA((2,)),
        ],
    )


@functools.lru_cache(maxsize=None)
def _scatter_call():
    qs = jax.ShapeDtypeStruct((NPAD, HQ), jnp.float32)
    return pl.kernel(
        _sc_scatter,
        out_type=[qs, qs, qs, qs],
        mesh=_sc_mesh(),
        compiler_params=pltpu.CompilerParams(use_tc_tiling_on_sc=False),
        scratch_types=[
            pltpu.VMEM((GRP, 2, 128), jnp.int32),
            pltpu.VMEM((GRP, 2, 128), jnp.int32),
            pltpu.VMEM((GRP, 128, HQ), jnp.float32),
            pltpu.VMEM((GRP, 128, HQ), jnp.float32),
            pltpu.VMEM((INIT_CHUNK, HQ), jnp.float32),
            pltpu.VMEM_SHARED((ACC_ROWS, HQ), jnp.float32),
            pltpu.SemaphoreType.DMA((2,)),
            pltpu.SemaphoreType.DMA((2,)),
            pltpu.SemaphoreType.DMA((2,)),
        ],
    )


def _tc_first(dga_ref, dgb_ref, x_ref, w_ref, u0_ref, u1_ref, u2_ref, u3_ref,
              dinv_ref):
    deg = dga_ref[:, 0] + dgb_ref[:, 0] - 1.0
    dinv = (1.0 / jnp.sqrt(deg))[:, None]
    t = jnp.dot(x_ref[...], w_ref[...], preferred_element_type=jnp.float32)
    u = t * dinv
    u0_ref[...] = u[:, 0 * HQ:1 * HQ]
    u1_ref[...] = u[:, 1 * HQ:2 * HQ]
    u2_ref[...] = u[:, 2 * HQ:3 * HQ]
    u3_ref[...] = u[:, 3 * HQ:4 * HQ]
    dinv_ref[...] = dinv


def _tc_layer(s0_ref, s1_ref, s2_ref, s3_ref, dinv_ref, b_ref, w_ref,
              u0_ref, u1_ref, u2_ref, u3_ref):
    dinv = dinv_ref[...]
    spu = jnp.concatenate(
        [s0_ref[...], s1_ref[...], s2_ref[...], s3_ref[...]], axis=1)
    h = jnp.maximum(spu * dinv + b_ref[...], 0.0)
    u = jnp.dot(h, w_ref[...], preferred_element_type=jnp.float32) * dinv
    u0_ref[...] = u[:, 0 * HQ:1 * HQ]
    u1_ref[...] = u[:, 1 * HQ:2 * HQ]
    u2_ref[...] = u[:, 2 * HQ:3 * HQ]
    u3_ref[...] = u[:, 3 * HQ:4 * HQ]


def _tc_pool(s0_ref, s1_ref, s2_ref, s3_ref, dinv_ref, b_ref, batch_ref,
             sums_ref, cnt_ref):
    i = pl.program_id(0)

    @pl.when(i == 0)
    def _():
        sums_ref[...] = jnp.zeros_like(sums_ref)
        cnt_ref[...] = jnp.zeros_like(cnt_ref)

    dinv = dinv_ref[...]
    spu = jnp.concatenate(
        [s0_ref[...], s1_ref[...], s2_ref[...], s3_ref[...]], axis=1)
    h = jnp.maximum(spu * dinv + b_ref[...], 0.0)
    seg = lax.broadcasted_iota(jnp.int32, (P, BLK), 0)
    onehot = (seg == batch_ref[0, 0, :][None, :]).astype(jnp.float32)
    sums_ref[...] += jnp.dot(onehot, h, preferred_element_type=jnp.float32)
    cnt_ref[...] += jnp.sum(onehot, axis=1, keepdims=True) * jnp.ones(
        (1, 8), jnp.float32)


def _tc_stage2(sums_ref, cnt_ref, iei_ref,
               wi1_ref, bi1_ref, wi2_ref, bi2_ref,
               wp1_ref, bp1_ref, wp2_ref, bp2_ref, out_ref):
    crs = []
    for c in range(C):
        poly = (sums_ref[c * P_PER:(c + 1) * P_PER, :]
                / jnp.maximum(cnt_ref[c * P_PER:(c + 1) * P_PER, :1], 1.0))
        src = iei_ref[c, 0, :]
        dst = iei_ref[c, 1, :]
        io = lax.broadcasted_iota(jnp.int32, (P_PER, E_INTER), 0)
        oh_s = (io == src[None, :]).astype(jnp.float32)
        oh_d = (io == dst[None, :]).astype(jnp.float32)
        adj = lax.dot_general(oh_d, oh_s, (((1,), (1,)), ((), ())),
                              preferred_element_type=jnp.float32)
        deg = 1.0 + jnp.sum(adj, axis=1)
        dv = (1.0 / jnp.sqrt(deg))[:, None]
        nadj = dv * adj * dv[:, 0][None, :] + jnp.diag((dv * dv)[:, 0])

        def conv(px, w_ref, b_ref):
            t = jnp.dot(px, w_ref[...], preferred_element_type=jnp.float32)
            return jnp.dot(nadj, t, preferred_element_type=jnp.float32) + b_ref[...]

        h = jnp.maximum(conv(poly, wi1_ref, bi1_ref), 0.0)
        h = jnp.maximum(conv(h, wi2_ref, bi2_ref), 0.0)
        crs.append(jnp.mean(h, axis=0)[None, :])
    cr = jnp.concatenate(crs, axis=0)
    z = jnp.maximum(jnp.dot(cr, wp1_ref[...],
                            preferred_element_type=jnp.float32) + bp1_ref[...],
                    0.0)
    out_ref[...] = jnp.dot(z, wp2_ref[...],
                           preferred_element_type=jnp.float32) + bp2_ref[...]


def _row_spec(shape):
    return pl.BlockSpec(shape, lambda i: (i, 0))


def _full_spec(shape):
    return pl.BlockSpec(shape, lambda i: (0, 0))


_q_blocks = [_row_spec((BLK, HQ)) for _ in range(4)]
_q_shapes = [jax.ShapeDtypeStruct((NPAD, HQ), jnp.float32) for _ in range(4)]

_first_call = pl.pallas_call(
    _tc_first,
    grid=(NB,),
    in_specs=[_row_spec((BLK, HQ)), _row_spec((BLK, HQ)),
              _row_spec((BLK, D_IN)), _full_spec((D_IN, H))],
    out_specs=_q_blocks + [_row_spec((BLK, 1))],
    out_shape=_q_shapes + [jax.ShapeDtypeStruct((N, 1), jnp.float32)],
)

_layer_call = pl.pallas_call(
    _tc_layer,
    grid=(NB,),
    in_specs=_q_blocks + [_row_spec((BLK, 1)), _full_spec((1, H)),
                          _full_spec((H, H))],
    out_specs=_q_blocks,
    out_shape=_q_shapes,
)

_pool_call = pl.pallas_call(
    _tc_pool,
    grid=(NB,),
    in_specs=_q_blocks + [_row_spec((BLK, 1)), _full_spec((1, H)),
                          pl.BlockSpec((1, 1, BLK), lambda i: (i, 0, 0))],
    out_specs=[_full_spec((P, H)), _full_spec((P, 8))],
    out_shape=[jax.ShapeDtypeStruct((P, H), jnp.float32),
               jax.ShapeDtypeStruct((P, 8), jnp.float32)],
)

_stage2_call = pl.pallas_call(
    _tc_stage2,
    grid=(1,),
    in_specs=[_full_spec((P, H)), _full_spec((P, 8)),
              pl.BlockSpec((C, 2, E_INTER), lambda i: (0, 0, 0)),
              _full_spec((H, H)), _full_spec((1, H)),
              _full_spec((H, H)), _full_spec((1, H)),
              _full_spec((H, H)), _full_spec((1, H)),
              _full_spec((H, 1)), _full_spec((1, 1))],
    out_specs=[_full_spec((C, 1))],
    out_shape=[jax.ShapeDtypeStruct((C, 1), jnp.float32)],
)


@jax.jit
def kernel(x, edge_index, batch, inter_edge_index,
           W_intra1, b_intra1, W_intra2, b_intra2, W_intra3, b_intra3,
           W_inter1, b_inter1, W_inter2, b_inter2,
           W_p1, b_p1, W_p2, b_p2):
    pad = EPAD - E
    padv = jnp.stack([jnp.zeros((pad,), jnp.int32),
                      jnp.full((pad,), DUMMY, jnp.int32)])
    edges = jnp.concatenate([edge_index, padv], axis=1).reshape(
        2, IDXROWS, 128).transpose(1, 0, 2)
    ones_i = jnp.ones((INIT_CHUNK, HQ), jnp.float32)

    dga, dgb = _deg_call()(ones_i, edges)
    *u, dinv = _first_call(dga, dgb, x, W_intra1)
    s = _scatter_call()(*u, edges)
    u = _layer_call(*s, dinv, b_intra1.reshape(1, H), W_intra2)
    s = _scatter_call()(*u, edges)
    u = _layer_call(*s, dinv, b_intra2.reshape(1, H), W_intra3)
    s = _scatter_call()(*u, edges)
    sums, cnt = _pool_call(*s, dinv, b_intra3.reshape(1, H),
                           batch.reshape(NB, 1, BLK))
    pred = _stage2_call(
        sums, cnt, inter_edge_index,
        W_inter1, b_inter1.reshape(1, H), W_inter2, b_inter2.reshape(1, H),
        W_p1, b_p1.reshape(1, H), W_p2, b_p2.reshape(1, 1))
    return pred[0]
